# baseline (device time: 198478 ns/iter reference)
import jax
import jax.numpy as jnp
from jax import lax
from jax.experimental import pallas as pl
from jax.experimental.pallas import tpu as pltpu

N_DEV = 4
M, K_SHARD, N = 4096, 1024, 2048
M_CHUNK = M // N_DEV
N_HALF = N // 2
N_HOPS = 2 * (N_DEV - 1)
S = 4
W = N_HALF // S


def kernel(x, w_mat):
    x = x.astype(jnp.bfloat16)
    w = w_mat.astype(jnp.bfloat16)

    def body(x_ref, w_ref, out_ref, recv_r, recv_l, stage_r, stage_l,
             ssem_r, rsem_r, ssem_l, rsem_l, credit_r, credit_l):
        my = lax.axis_index("i")
        left = jnp.mod(my + N_DEV - 1, N_DEV)
        right = jnp.mod(my + 1, N_DEV)

        lanes = []
        for k in range(S):
            lanes.append(dict(d=0, col=k * W, lcol=k * W, to=right,
                              frm=left, recv=recv_r, stage=stage_r,
                              ssem=ssem_r.at[k], rsem=rsem_r.at[k],
                              credit=credit_r.at[k]))
            lanes.append(dict(d=1, col=N_HALF + k * W, lcol=k * W, to=left,
                              frm=right, recv=recv_l, stage=stage_l,
                              ssem=ssem_l.at[k], rsem=rsem_l.at[k],
                              credit=credit_l.at[k]))

        def send_chunk(d, h):
            return jnp.mod(my + 3 - h + 8, N_DEV) if d == 0 else \
                jnp.mod(my + 1 + h, N_DEV)

        def recv_chunk(d, h):
            return jnp.mod(my + 2 - h + 8, N_DEV) if d == 0 else \
                jnp.mod(my + 2 + h, N_DEV)

        def strip(c, col):
            return out_ref[pl.ds(c * M_CHUNK, M_CHUNK), pl.ds(col, W)]

        def set_strip(c, col, val):
            out_ref[pl.ds(c * M_CHUNK, M_CHUNK), pl.ds(col, W)] = val

        def dot_cols(c, col0, ncol, to_ref=None):
            val = jnp.dot(
                x_ref[pl.ds(c * M_CHUNK, M_CHUNK), :],
                w_ref[:, pl.ds(col0, ncol)],
                preferred_element_type=jnp.float32,
            )
            if to_ref is None:
                out_ref[pl.ds(c * M_CHUNK, M_CHUNK), pl.ds(col0, ncol)] = val
            else:
                to_ref[:, :] = val.astype(jnp.bfloat16)

        barrier_sem = pltpu.get_barrier_semaphore()
        for nbr in (left, right):
            pl.semaphore_signal(
                barrier_sem, inc=1,
                device_id=(nbr,), device_id_type=pl.DeviceIdType.MESH,
            )
        pl.semaphore_wait(barrier_sem, 2)

        dot_cols(jnp.mod(my + 3, N_DEV), 0, N_HALF, to_ref=stage_r)
        dot_cols(jnp.mod(my + 1, N_DEV), N_HALF, N_HALF, to_ref=stage_l)

        def issue(ln, h):
            src = (
                ln["stage"].at[:, pl.ds(ln["lcol"], W)] if h == 0
                else ln["recv"].at[(h - 1) % 2, :, pl.ds(ln["lcol"], W)]
            )
            rdma = pltpu.make_async_remote_copy(
                src_ref=src,
                dst_ref=ln["recv"].at[h % 2, :, pl.ds(ln["lcol"], W)],
                send_sem=ln["ssem"].at[h % 2],
                recv_sem=ln["rsem"].at[h % 2],
                device_id=(ln["to"],),
                device_id_type=pl.DeviceIdType.MESH,
            )
            rdma.start()
            return rdma

        pending = {}
        for ln in lanes:
            pending[id(ln)] = issue(ln, 0)

        dot_cols(jnp.mod(my + 2, N_DEV), 0, N)

        for h in range(N_HOPS):
            slot = h % 2
            for ln in lanes:
                d = ln["d"]
                rdma = pending[id(ln)]
                rdma.wait_recv()
                rdma.wait_send()
                if 1 <= h <= 4:
                    pl.semaphore_signal(
                        ln["credit"], inc=1,
                        device_id=(ln["frm"],),
                        device_id_type=pl.DeviceIdType.MESH,
                    )
                c = recv_chunk(d, h)
                if h <= 2:
                    if h == 2:
                        set_strip(
                            c, ln["col"],
                            strip(c, ln["col"])
                            + ln["recv"][slot, :, pl.ds(ln["lcol"], W)]
                            .astype(jnp.float32),
                        )
                        ln["recv"][slot, :, pl.ds(ln["lcol"], W)] = (
                            strip(c, ln["col"]).astype(jnp.bfloat16)
                        )
                    else:
                        ln["recv"][slot, :, pl.ds(ln["lcol"], W)] = (
                            (
                                ln["recv"][slot, :, pl.ds(ln["lcol"], W)]
                                .astype(jnp.float32)
                                + strip(c, ln["col"])
                            ).astype(jnp.bfloat16)
                        )
                if h < N_HOPS - 1:
                    if h + 1 >= 2:
                        pl.semaphore_wait(ln["credit"], 1)
                    pending[id(ln)] = issue(ln, h + 1)
                if h >= 3:
                    set_strip(
                        c, ln["col"],
                        ln["recv"][slot, :, pl.ds(ln["lcol"], W)]
                        .astype(jnp.float32),
                    )
            if h == 0:
                dot_cols(jnp.mod(my + 1, N_DEV), 0, N_HALF)
                dot_cols(jnp.mod(my + 3, N_DEV), N_HALF, N_HALF)
            if h == 1:
                dot_cols(my, 0, N)

    return pl.pallas_call(
        body,
        out_shape=jax.ShapeDtypeStruct((M, N), jnp.float32),
        in_specs=[
            pl.BlockSpec(memory_space=pltpu.VMEM),
            pl.BlockSpec(memory_space=pltpu.VMEM),
        ],
        out_specs=pl.BlockSpec(memory_space=pltpu.VMEM),
        scratch_shapes=[
            pltpu.VMEM((2, M_CHUNK, N_HALF), jnp.bfloat16),
            pltpu.VMEM((2, M_CHUNK, N_HALF), jnp.bfloat16),
            pltpu.VMEM((M_CHUNK, N_HALF), jnp.bfloat16),
            pltpu.VMEM((M_CHUNK, N_HALF), jnp.bfloat16),
            pltpu.SemaphoreType.DMA((S, 2)),
            pltpu.SemaphoreType.DMA((S, 2)),
            pltpu.SemaphoreType.DMA((S, 2)),
            pltpu.SemaphoreType.DMA((S, 2)),
            pltpu.SemaphoreType.REGULAR((S,)),
            pltpu.SemaphoreType.REGULAR((S,)),
        ],
        compiler_params=pltpu.CompilerParams(
            collective_id=0,
            vmem_limit_bytes=64 * 1024 * 1024,
        ),
    )(x, w)


# device time: 188680 ns/iter; 1.0519x vs baseline; 1.0519x over previous
import jax
import jax.numpy as jnp
from jax import lax
from jax.experimental import pallas as pl
from jax.experimental.pallas import tpu as pltpu

N_DEV = 4
M, K_SHARD, N = 4096, 1024, 2048
M_CHUNK = M // N_DEV
N_HALF = N // 2
N_HOPS = 2 * (N_DEV - 1)
S = 2
W = N_HALF // S


def kernel(x, w_mat):
    w = w_mat.astype(jnp.bfloat16)

    def body(x_ref, w_ref, out_ref, xbuf, recv_r, recv_l, stage_r, stage_l,
             xsem, ssem_r, rsem_r, ssem_l, rsem_l, credit_r, credit_l):
        my = lax.axis_index("i")
        left = jnp.mod(my + N_DEV - 1, N_DEV)
        right = jnp.mod(my + 1, N_DEV)

        lanes = []
        for k in range(S):
            lanes.append(dict(d=0, col=k * W, lcol=k * W, to=right,
                              frm=left, recv=recv_r, stage=stage_r,
                              ssem=ssem_r.at[k], rsem=rsem_r.at[k],
                              credit=credit_r.at[k]))
            lanes.append(dict(d=1, col=N_HALF + k * W, lcol=k * W, to=left,
                              frm=right, recv=recv_l, stage=stage_l,
                              ssem=ssem_l.at[k], rsem=rsem_l.at[k],
                              credit=credit_l.at[k]))

        def send_chunk(d, h):
            return jnp.mod(my + 3 - h + 8, N_DEV) if d == 0 else \
                jnp.mod(my + 1 + h, N_DEV)

        def recv_chunk(d, h):
            return jnp.mod(my + 2 - h + 8, N_DEV) if d == 0 else \
                jnp.mod(my + 2 + h, N_DEV)

        def strip(c, col):
            return out_ref[pl.ds(c * M_CHUNK, M_CHUNK), pl.ds(col, W)]

        def set_strip(c, col, val):
            out_ref[pl.ds(c * M_CHUNK, M_CHUNK), pl.ds(col, W)] = val

        def xload(c, xslot):
            cp = pltpu.make_async_copy(
                x_ref.at[pl.ds(c * M_CHUNK, M_CHUNK), :],
                xbuf.at[xslot],
                xsem.at[xslot],
            )
            cp.start()
            return cp

        def dot_cols(xslot, c, col0, ncol, to_ref=None):
            val = jnp.dot(
                xbuf[xslot].astype(jnp.bfloat16),
                w_ref[:, pl.ds(col0, ncol)],
                preferred_element_type=jnp.float32,
            )
            if to_ref is None:
                out_ref[pl.ds(c * M_CHUNK, M_CHUNK), pl.ds(col0, ncol)] = val
            else:
                to_ref[:, :] = val.astype(jnp.bfloat16)

        barrier_sem = pltpu.get_barrier_semaphore()
        for nbr in (left, right):
            pl.semaphore_signal(
                barrier_sem, inc=1,
                device_id=(nbr,), device_id_type=pl.DeviceIdType.MESH,
            )
        pl.semaphore_wait(barrier_sem, 2)

        x0 = xload(jnp.mod(my + 3, N_DEV), 0)
        x1 = xload(jnp.mod(my + 1, N_DEV), 1)
        x0.wait()
        dot_cols(0, jnp.mod(my + 3, N_DEV), 0, N_HALF, to_ref=stage_r)
        x1.wait()
        dot_cols(1, jnp.mod(my + 1, N_DEV), N_HALF, N_HALF, to_ref=stage_l)

        def issue(ln, h):
            src = (
                ln["stage"].at[:, pl.ds(ln["lcol"], W)] if h == 0
                else ln["recv"].at[(h - 1) % 2, :, pl.ds(ln["lcol"], W)]
            )
            rdma = pltpu.make_async_remote_copy(
                src_ref=src,
                dst_ref=ln["recv"].at[h % 2, :, pl.ds(ln["lcol"], W)],
                send_sem=ln["ssem"].at[h % 2],
                recv_sem=ln["rsem"].at[h % 2],
                device_id=(ln["to"],),
                device_id_type=pl.DeviceIdType.MESH,
            )
            rdma.start()
            return rdma

        pending = {}
        for ln in lanes:
            pending[id(ln)] = issue(ln, 0)

        dot_cols(1, jnp.mod(my + 1, N_DEV), 0, N_HALF)
        dot_cols(0, jnp.mod(my + 3, N_DEV), N_HALF, N_HALF)
        x0 = xload(jnp.mod(my + 2, N_DEV), 0)
        x1 = xload(my, 1)
        x0.wait()
        dot_cols(0, jnp.mod(my + 2, N_DEV), 0, N)

        for h in range(N_HOPS):
            slot = h % 2
            for ln in lanes:
                d = ln["d"]
                rdma = pending[id(ln)]
                rdma.wait_recv()
                rdma.wait_send()
                if 1 <= h <= 4:
                    pl.semaphore_signal(
                        ln["credit"], inc=1,
                        device_id=(ln["frm"],),
                        device_id_type=pl.DeviceIdType.MESH,
                    )
                c = recv_chunk(d, h)
                if h <= 2:
                    if h == 2:
                        set_strip(
                            c, ln["col"],
                            strip(c, ln["col"])
                            + ln["recv"][slot, :, pl.ds(ln["lcol"], W)]
                            .astype(jnp.float32),
                        )
                        ln["recv"][slot, :, pl.ds(ln["lcol"], W)] = (
                            strip(c, ln["col"]).astype(jnp.bfloat16)
                        )
                    else:
                        ln["recv"][slot, :, pl.ds(ln["lcol"], W)] = (
                            (
                                ln["recv"][slot, :, pl.ds(ln["lcol"], W)]
                                .astype(jnp.float32)
                                + strip(c, ln["col"])
                            ).astype(jnp.bfloat16)
                        )
                if h < N_HOPS - 1:
                    if h + 1 >= 2:
                        pl.semaphore_wait(ln["credit"], 1)
                    pending[id(ln)] = issue(ln, h + 1)
                if h >= 3:
                    set_strip(
                        c, ln["col"],
                        ln["recv"][slot, :, pl.ds(ln["lcol"], W)]
                        .astype(jnp.float32),
                    )
            if h == 1:
                x1.wait()
                dot_cols(1, my, 0, N)

    return pl.pallas_call(
        body,
        out_shape=jax.ShapeDtypeStruct((M, N), jnp.float32),
        in_specs=[
            pl.BlockSpec(memory_space=pl.ANY),
            pl.BlockSpec(memory_space=pltpu.VMEM),
        ],
        out_specs=pl.BlockSpec(memory_space=pltpu.VMEM),
        scratch_shapes=[
            pltpu.VMEM((2, M_CHUNK, K_SHARD), jnp.float32),
            pltpu.VMEM((2, M_CHUNK, N_HALF), jnp.bfloat16),
            pltpu.VMEM((2, M_CHUNK, N_HALF), jnp.bfloat16),
            pltpu.VMEM((M_CHUNK, N_HALF), jnp.bfloat16),
            pltpu.VMEM((M_CHUNK, N_HALF), jnp.bfloat16),
            pltpu.SemaphoreType.DMA((2,)),
            pltpu.SemaphoreType.DMA((S, 2)),
            pltpu.SemaphoreType.DMA((S, 2)),
            pltpu.SemaphoreType.DMA((S, 2)),
            pltpu.SemaphoreType.DMA((S, 2)),
            pltpu.SemaphoreType.REGULAR((S,)),
            pltpu.SemaphoreType.REGULAR((S,)),
        ],
        compiler_params=pltpu.CompilerParams(
            collective_id=0,
            vmem_limit_bytes=64 * 1024 * 1024,
        ),
    )(x, w)


# device time: 179707 ns/iter; 1.1045x vs baseline; 1.0499x over previous
import jax
import jax.numpy as jnp
from jax import lax
from jax.experimental import pallas as pl
from jax.experimental.pallas import tpu as pltpu

N_DEV = 4
M, K_SHARD, N = 4096, 1024, 2048
M_CHUNK = M // N_DEV
N_HALF = N // 2
N_HOPS = 2 * (N_DEV - 1)
S = 2
W = N_HALF // S


def kernel(x, w_mat):
    w = w_mat.astype(jnp.bfloat16)

    def body(x_ref, w_ref, out_ref, xbuf, abuf, recv_r, recv_l, vstage,
             xsem, stsem, ssem_r, rsem_r, ssem_l, rsem_l,
             credit_r, credit_l):
        my = lax.axis_index("i")
        left = jnp.mod(my + N_DEV - 1, N_DEV)
        right = jnp.mod(my + 1, N_DEV)

        lanes = []
        for k in range(S):
            lanes.append(dict(d=0, i=2 * k, col=k * W, lcol=k * W,
                              to=right, frm=left, recv=recv_r,
                              ssem=ssem_r.at[k], rsem=rsem_r.at[k],
                              credit=credit_r.at[k]))
            lanes.append(dict(d=1, i=2 * k + 1, col=N_HALF + k * W,
                              lcol=k * W, to=left, frm=right, recv=recv_l,
                              ssem=ssem_l.at[k], rsem=rsem_l.at[k],
                              credit=credit_l.at[k]))

        def send_chunk(d, h):
            return jnp.mod(my + 3 - h + 8, N_DEV) if d == 0 else \
                jnp.mod(my + 1 + h, N_DEV)

        def recv_chunk(d, h):
            return jnp.mod(my + 2 - h + 8, N_DEV) if d == 0 else \
                jnp.mod(my + 2 + h, N_DEV)

        def xload(c, xslot):
            cp = pltpu.make_async_copy(
                x_ref.at[pl.ds(c * M_CHUNK, M_CHUNK), :],
                xbuf.at[xslot],
                xsem.at[xslot],
            )
            cp.start()
            return cp

        def dot_cols(xslot, c, col0, ncol):
            abuf[c, :, pl.ds(col0, ncol)] = jnp.dot(
                xbuf[xslot].astype(jnp.bfloat16),
                w_ref[:, pl.ds(col0, ncol)],
                preferred_element_type=jnp.float32,
            ).astype(jnp.bfloat16)

        barrier_sem = pltpu.get_barrier_semaphore()
        for nbr in (left, right):
            pl.semaphore_signal(
                barrier_sem, inc=1,
                device_id=(nbr,), device_id_type=pl.DeviceIdType.MESH,
            )
        pl.semaphore_wait(barrier_sem, 2)

        c_r0 = jnp.mod(my + 3, N_DEV)
        c_l0 = jnp.mod(my + 1, N_DEV)
        x0 = xload(c_r0, 0)
        x1 = xload(c_l0, 1)
        x0.wait()
        dot_cols(0, c_r0, 0, N_HALF)
        x1.wait()
        dot_cols(1, c_l0, N_HALF, N_HALF)

        def issue(ln, h):
            src = (
                abuf.at[send_chunk(ln["d"], 0), :, pl.ds(ln["col"], W)]
                if h == 0
                else ln["recv"].at[(h - 1) % 2, :, pl.ds(ln["lcol"], W)]
            )
            rdma = pltpu.make_async_remote_copy(
                src_ref=src,
                dst_ref=ln["recv"].at[h % 2, :, pl.ds(ln["lcol"], W)],
                send_sem=ln["ssem"].at[h % 2],
                recv_sem=ln["rsem"].at[h % 2],
                device_id=(ln["to"],),
                device_id_type=pl.DeviceIdType.MESH,
            )
            rdma.start()
            return rdma

        pending = {}
        for ln in lanes:
            pending[id(ln)] = issue(ln, 0)

        dot_cols(1, c_l0, 0, N_HALF)
        dot_cols(0, c_r0, N_HALF, N_HALF)
        x0 = xload(jnp.mod(my + 2, N_DEV), 0)
        x1 = xload(my, 1)
        x0.wait()
        dot_cols(0, jnp.mod(my + 2, N_DEV), 0, N)

        stores = {}

        def store(ln, c, slot):
            if id(ln) in stores:
                stores[id(ln)].wait()
            vstage[ln["i"], :, :] = (
                ln["recv"][slot, :, pl.ds(ln["lcol"], W)]
                .astype(jnp.float32)
            )
            cp = pltpu.make_async_copy(
                vstage.at[ln["i"]],
                out_ref.at[pl.ds(c * M_CHUNK, M_CHUNK),
                           pl.ds(ln["col"], W)],
                stsem.at[ln["i"]],
            )
            cp.start()
            stores[id(ln)] = cp

        for h in range(N_HOPS):
            slot = h % 2
            for ln in lanes:
                d = ln["d"]
                rdma = pending[id(ln)]
                rdma.wait_recv()
                rdma.wait_send()
                if 1 <= h <= 4:
                    pl.semaphore_signal(
                        ln["credit"], inc=1,
                        device_id=(ln["frm"],),
                        device_id_type=pl.DeviceIdType.MESH,
                    )
                c = recv_chunk(d, h)
                if h <= 2:
                    ln["recv"][slot, :, pl.ds(ln["lcol"], W)] = (
                        ln["recv"][slot, :, pl.ds(ln["lcol"], W)]
                        + abuf[c, :, pl.ds(ln["col"], W)]
                    )
                if h < N_HOPS - 1:
                    if h + 1 >= 2:
                        pl.semaphore_wait(ln["credit"], 1)
                    pending[id(ln)] = issue(ln, h + 1)
                if h >= 2:
                    store(ln, c, slot)
            if h == 1:
                x1.wait()
                dot_cols(1, my, 0, N)

        for ln in lanes:
            stores[id(ln)].wait()

    return pl.pallas_call(
        body,
        out_shape=jax.ShapeDtypeStruct((M, N), jnp.float32),
        in_specs=[
            pl.BlockSpec(memory_space=pl.ANY),
            pl.BlockSpec(memory_space=pltpu.VMEM),
        ],
        out_specs=pl.BlockSpec(memory_space=pl.ANY),
        scratch_shapes=[
            pltpu.VMEM((2, M_CHUNK, K_SHARD), jnp.float32),
            pltpu.VMEM((N_DEV, M_CHUNK, N), jnp.bfloat16),
            pltpu.VMEM((2, M_CHUNK, N_HALF), jnp.bfloat16),
            pltpu.VMEM((2, M_CHUNK, N_HALF), jnp.bfloat16),
            pltpu.VMEM((2 * S, M_CHUNK, W), jnp.float32),
            pltpu.SemaphoreType.DMA((2,)),
            pltpu.SemaphoreType.DMA((2 * S,)),
            pltpu.SemaphoreType.DMA((S, 2)),
            pltpu.SemaphoreType.DMA((S, 2)),
            pltpu.SemaphoreType.DMA((S, 2)),
            pltpu.SemaphoreType.DMA((S, 2)),
            pltpu.SemaphoreType.REGULAR((S,)),
            pltpu.SemaphoreType.REGULAR((S,)),
        ],
        compiler_params=pltpu.CompilerParams(
            collective_id=0,
            vmem_limit_bytes=64 * 1024 * 1024,
        ),
    )(x, w)


# device time: 173789 ns/iter; 1.1421x vs baseline; 1.0341x over previous
import jax
import jax.numpy as jnp
from jax import lax
from jax.experimental import pallas as pl
from jax.experimental.pallas import tpu as pltpu

N_DEV = 4
M, K_SHARD, N = 4096, 1024, 2048
M_CHUNK = M // N_DEV
N_HALF = N // 2
N_HOPS = 2 * (N_DEV - 1)
S = 2
W = N_HALF // S


def kernel(x, w_mat):
    def body(x_ref, w_ref, out_ref, xbuf, wbuf, wbf, abuf, recv_r, recv_l,
             vstage, xsem, wsem, stsem, ssem_r, rsem_r, ssem_l, rsem_l,
             credit_r, credit_l):
        my = lax.axis_index("i")
        left = jnp.mod(my + N_DEV - 1, N_DEV)
        right = jnp.mod(my + 1, N_DEV)

        lanes = []
        for k in range(S):
            lanes.append(dict(d=0, i=2 * k, col=k * W, lcol=k * W,
                              to=right, frm=left, recv=recv_r,
                              ssem=ssem_r.at[k], rsem=rsem_r.at[k],
                              credit=credit_r.at[k]))
            lanes.append(dict(d=1, i=2 * k + 1, col=N_HALF + k * W,
                              lcol=k * W, to=left, frm=right, recv=recv_l,
                              ssem=ssem_l.at[k], rsem=rsem_l.at[k],
                              credit=credit_l.at[k]))

        def send_chunk(d, h):
            return jnp.mod(my + 3 - h + 8, N_DEV) if d == 0 else \
                jnp.mod(my + 1 + h, N_DEV)

        def recv_chunk(d, h):
            return jnp.mod(my + 2 - h + 8, N_DEV) if d == 0 else \
                jnp.mod(my + 2 + h, N_DEV)

        def xload(c, xslot):
            cp = pltpu.make_async_copy(
                x_ref.at[pl.ds(c * M_CHUNK, M_CHUNK), :],
                xbuf.at[xslot],
                xsem.at[xslot],
            )
            cp.start()
            return cp

        def dot_cols(xslot, c, col0, ncol):
            abuf[c, :, pl.ds(col0, ncol)] = jnp.dot(
                xbuf[xslot].astype(jnp.bfloat16),
                wbf[:, pl.ds(col0, ncol)],
                preferred_element_type=jnp.float32,
            ).astype(jnp.bfloat16)

        c_r0 = jnp.mod(my + 3, N_DEV)
        c_l0 = jnp.mod(my + 1, N_DEV)
        wload = pltpu.make_async_copy(w_ref, wbuf, wsem)
        wload.start()
        x0 = xload(c_r0, 0)
        x1 = xload(c_l0, 1)

        barrier_sem = pltpu.get_barrier_semaphore()
        for nbr in (left, right):
            pl.semaphore_signal(
                barrier_sem, inc=1,
                device_id=(nbr,), device_id_type=pl.DeviceIdType.MESH,
            )
        pl.semaphore_wait(barrier_sem, 2)

        wload.wait()
        wbf[:, :] = wbuf[:, :].astype(jnp.bfloat16)

        x0.wait()
        dot_cols(0, c_r0, 0, N_HALF)
        x1.wait()
        dot_cols(1, c_l0, N_HALF, N_HALF)

        def issue(ln, h):
            src = (
                abuf.at[send_chunk(ln["d"], 0), :, pl.ds(ln["col"], W)]
                if h == 0
                else ln["recv"].at[(h - 1) % 2, :, pl.ds(ln["lcol"], W)]
            )
            rdma = pltpu.make_async_remote_copy(
                src_ref=src,
                dst_ref=ln["recv"].at[h % 2, :, pl.ds(ln["lcol"], W)],
                send_sem=ln["ssem"].at[h % 2],
                recv_sem=ln["rsem"].at[h % 2],
                device_id=(ln["to"],),
                device_id_type=pl.DeviceIdType.MESH,
            )
            rdma.start()
            return rdma

        pending = {}
        for ln in lanes:
            pending[id(ln)] = issue(ln, 0)

        dot_cols(1, c_l0, 0, N_HALF)
        dot_cols(0, c_r0, N_HALF, N_HALF)
        x0 = xload(jnp.mod(my + 2, N_DEV), 0)
        x1 = xload(my, 1)
        x0.wait()
        dot_cols(0, jnp.mod(my + 2, N_DEV), 0, N)

        stores = {}

        def store(ln, c, slot):
            if id(ln) in stores:
                stores[id(ln)].wait()
            vstage[ln["i"], :, :] = (
                ln["recv"][slot, :, pl.ds(ln["lcol"], W)]
                .astype(jnp.float32)
            )
            cp = pltpu.make_async_copy(
                vstage.at[ln["i"]],
                out_ref.at[pl.ds(c * M_CHUNK, M_CHUNK),
                           pl.ds(ln["col"], W)],
                stsem.at[ln["i"]],
            )
            cp.start()
            stores[id(ln)] = cp

        for h in range(N_HOPS):
            slot = h % 2
            for ln in lanes:
                d = ln["d"]
                rdma = pending[id(ln)]
                rdma.wait_recv()
                rdma.wait_send()
                if 1 <= h <= 4:
                    pl.semaphore_signal(
                        ln["credit"], inc=1,
                        device_id=(ln["frm"],),
                        device_id_type=pl.DeviceIdType.MESH,
                    )
                c = recv_chunk(d, h)
                if h <= 2:
                    ln["recv"][slot, :, pl.ds(ln["lcol"], W)] = (
                        ln["recv"][slot, :, pl.ds(ln["lcol"], W)]
                        + abuf[c, :, pl.ds(ln["col"], W)]
                    )
                if h < N_HOPS - 1:
                    if h + 1 >= 2:
                        pl.semaphore_wait(ln["credit"], 1)
                    pending[id(ln)] = issue(ln, h + 1)
                if h >= 2:
                    store(ln, c, slot)
            if h == 1:
                x1.wait()
                dot_cols(1, my, 0, N)

        for ln in lanes:
            stores[id(ln)].wait()

    return pl.pallas_call(
        body,
        out_shape=jax.ShapeDtypeStruct((M, N), jnp.float32),
        in_specs=[
            pl.BlockSpec(memory_space=pl.ANY),
            pl.BlockSpec(memory_space=pl.ANY),
        ],
        out_specs=pl.BlockSpec(memory_space=pl.ANY),
        scratch_shapes=[
            pltpu.VMEM((2, M_CHUNK, K_SHARD), jnp.float32),
            pltpu.VMEM((K_SHARD, N), jnp.float32),
            pltpu.VMEM((K_SHARD, N), jnp.bfloat16),
            pltpu.VMEM((N_DEV, M_CHUNK, N), jnp.bfloat16),
            pltpu.VMEM((2, M_CHUNK, N_HALF), jnp.bfloat16),
            pltpu.VMEM((2, M_CHUNK, N_HALF), jnp.bfloat16),
            pltpu.VMEM((2 * S, M_CHUNK, W), jnp.float32),
            pltpu.SemaphoreType.DMA((2,)),
            pltpu.SemaphoreType.DMA,
            pltpu.SemaphoreType.DMA((2 * S,)),
            pltpu.SemaphoreType.DMA((S, 2)),
            pltpu.SemaphoreType.DMA((S, 2)),
            pltpu.SemaphoreType.DMA((S, 2)),
            pltpu.SemaphoreType.DMA((S, 2)),
            pltpu.SemaphoreType.REGULAR((S,)),
            pltpu.SemaphoreType.REGULAR((S,)),
        ],
        compiler_params=pltpu.CompilerParams(
            collective_id=0,
            vmem_limit_bytes=64 * 1024 * 1024,
        ),
    )(x, w_mat)
